# Initial kernel scaffold; baseline (speedup 1.0000x reference)
import jax
import jax.numpy as jnp
from jax.experimental import pallas as pl


def kernel(node_type, coeff_idx, var_idx, src, dst, type_table, coeff_table, var_table, init_w, init_b, agg_comm_w, agg_comm_b, agg_bin_w, agg_bin_b, wih_comm, whh_comm, bih_comm, bhh_comm, wih_bin, whh_bin, bih_bin, bhh_bin, wih_leaf, whh_leaf, bih_leaf, bhh_leaf, ln_g, ln_b, glob_w, glob_b):
    raise NotImplementedError("write your pallas kernel here")


# speedup vs baseline: 5.3009x; 5.3009x over previous
import functools

import jax
import jax.numpy as jnp
from jax import lax
from jax.experimental import pallas as pl
from jax.experimental.pallas import tpu as pltpu
from jax.experimental.pallas import tpu_sc as plsc

N = 10000
E = 320000
H = 128
NUM_ROUNDS = 6

NW = 32
NPW = 320
NPAD = NW * NPW
CH = 128
EPAD = E + CH
DUMP = NPW


def _sc_agg_body(h, srcp, dstl, eoffp, c0i, c1i,
                 out_s, out_m, out_c0, out_c1,
                 eoff_v, idx_v, loc_v, rows_v, acc_s, acc_m, cidx_v, crow_v,
                 sem):
    c = lax.axis_index("c")
    s = lax.axis_index("s")
    w = c * 16 + s
    n0 = w * NPW

    pltpu.sync_copy(eoffp, eoff_v)
    ev = eoff_v[pl.ds(w, 16)]
    e0 = ev[0]
    e1 = ev[1]
    e0a = (e0 // 8) * 8

    zf = jnp.zeros((16,), jnp.float32)
    ninf = jnp.full((16,), -jnp.inf, jnp.float32)

    def init_body(i, carry):
        for j in range(8):
            acc_s[i, pl.ds(j * 16, 16)] = zf
            acc_m[i, pl.ds(j * 16, 16)] = ninf
        return carry

    lax.fori_loop(0, NPW + 1, init_body, 0)

    nch = (e1 - e0a + CH - 1) // CH

    def chunk_body(k, carry):
        base = e0a + k * CH
        pltpu.sync_copy(srcp.at[pl.ds(base, CH)], idx_v)
        pltpu.sync_copy(dstl.at[pl.ds(base, CH)], loc_v)
        cp = pltpu.async_copy(h.at[idx_v], rows_v, sem)
        for i in range(CH // 16):
            eid = base + i * 16 + lax.iota(jnp.int32, 16)
            lv = loc_v[pl.ds(i * 16, 16)]
            ok = (eid >= e0) & (eid < e1)
            loc_v[pl.ds(i * 16, 16)] = jnp.where(ok, lv, DUMP)
        cp.wait()

        def blk_body(b, cc):
            lv = loc_v[pl.ds(b * 16, 16)]
            rbase = b * 16
            for t in range(16):
                local = lv[t]
                for j in range(8):
                    row = rows_v[rbase + t, pl.ds(j * 16, 16)]
                    plsc.addupdate(acc_s.at[local, pl.ds(j * 16, 16)], row)
                    mm = acc_m[local, pl.ds(j * 16, 16)]
                    acc_m[local, pl.ds(j * 16, 16)] = jnp.maximum(mm, row)
            return cc

        lax.fori_loop(0, CH // 16, blk_body, 0)
        return carry

    lax.fori_loop(0, nch, chunk_body, 0)

    pltpu.sync_copy(acc_s.at[pl.ds(0, NPW)], out_s.at[pl.ds(n0, NPW)])
    pltpu.sync_copy(acc_m.at[pl.ds(0, NPW)], out_m.at[pl.ds(n0, NPW)])

    for t in range(NPW // 64):
        off = n0 + t * 64
        pltpu.sync_copy(c0i.at[pl.ds(off, 64)], cidx_v)
        pltpu.async_copy(h.at[cidx_v], crow_v, sem).wait()
        pltpu.sync_copy(crow_v, out_c0.at[pl.ds(off, 64)])
        pltpu.sync_copy(c1i.at[pl.ds(off, 64)], cidx_v)
        pltpu.async_copy(h.at[cidx_v], crow_v, sem).wait()
        pltpu.sync_copy(crow_v, out_c1.at[pl.ds(off, 64)])


_f32 = jnp.float32

_sc_agg = functools.partial(
    pl.kernel,
    out_type=[jax.ShapeDtypeStruct((NPAD, H), _f32)] * 4,
    mesh=plsc.VectorSubcoreMesh(core_axis_name="c", subcore_axis_name="s"),
    scratch_types=[
        pltpu.VMEM((48,), jnp.int32),
        pltpu.VMEM((CH,), jnp.int32),
        pltpu.VMEM((CH,), jnp.int32),
        pltpu.VMEM((CH, H), _f32),
        pltpu.VMEM((NPW + 1, H), _f32),
        pltpu.VMEM((NPW + 1, H), _f32),
        pltpu.VMEM((64,), jnp.int32),
        pltpu.VMEM((64, H), _f32),
        pltpu.SemaphoreType.DMA,
    ],
)(_sc_agg_body)


BR = 1000


def _mm(a, b):
    return lax.dot_general(a, b, (((1,), (0,)), ((), ())),
                           precision=lax.Precision.HIGHEST,
                           preferred_element_type=_f32)


def _sigmoid(x):
    return 1.0 / (1.0 + jnp.exp(-x))


def _gru(x, h, wih, whh, bih, bhh):
    gi = _mm(x, wih) + bih
    gh = _mm(h, whh) + bhh
    r = _sigmoid(gi[:, 0:H] + gh[:, 0:H])
    z = _sigmoid(gi[:, H:2 * H] + gh[:, H:2 * H])
    nn_ = jnp.tanh(gi[:, 2 * H:3 * H] + r * gh[:, 2 * H:3 * H])
    return (1.0 - z) * nn_ + z * h


def _tc_round_body(h, sm, mx, c0, c1, meta, acw, acb, abw, abb,
                   wihc, whhc, bihc, bhhc, wihb, whhb, bihb, bhhb,
                   wihl, whhl, bihl, bhhl, lng, lnb, hout):
    cnt = meta[:, 0:1]
    invd = meta[:, 1:2]
    il = meta[:, 2:3]
    ic = meta[:, 3:4]
    ip = meta[:, 4:5]
    hv = h[...]
    mean = sm[...] * invd
    mxv = jnp.where(cnt > 0.0, mx[...], 0.0)
    aggc = _mm(jnp.concatenate([mean, mxv], axis=1), acw[...]) + acb[...]
    aggb = _mm(jnp.concatenate([c0[...], c1[...]], axis=1), abw[...]) + abb[...]
    agg = jnp.where(il > 0.0, 0.0,
                    jnp.where(ic > 0.0, aggc,
                              jnp.where(ip > 0.0, aggb, mean)))
    uc = _gru(agg, hv, wihc[...], whhc[...], bihc[...], bhhc[...])
    ub = _gru(agg, hv, wihb[...], whhb[...], bihb[...], bhhb[...])
    ul = _gru(agg, hv, wihl[...], whhl[...], bihl[...], bhhl[...])
    upd = jnp.where(il > 0.0, ul, jnp.where(ip > 0.0, ub, uc))
    x = upd + hv
    mu = jnp.mean(x, axis=1, keepdims=True)
    var = jnp.mean((x - mu) * (x - mu), axis=1, keepdims=True)
    hout[...] = (x - mu) * lax.rsqrt(var + 1e-5) * lng[...] + lnb[...]


def _node_spec():
    return pl.BlockSpec((BR, H), lambda i: (i, 0))


def _full_spec(shape):
    return pl.BlockSpec(shape, lambda i: tuple(0 for _ in shape))


def _tc_round(h, sm, mx, c0, c1, meta, *weights):
    wspecs = [_full_spec(w.shape) for w in weights]
    return pl.pallas_call(
        _tc_round_body,
        grid=(N // BR,),
        in_specs=[_node_spec()] * 5 + [pl.BlockSpec((BR, 8), lambda i: (i, 0))]
                 + wspecs,
        out_specs=_node_spec(),
        out_shape=jax.ShapeDtypeStruct((N, H), _f32),
    )(h, sm, mx, c0, c1, meta, *weights)


def _tc_init_body(meta, ttab, ctab, vtab, iw, ib, hout):
    nt = meta[:, 0:1]
    ci = meta[:, 1:2]
    vi = meta[:, 2:3]
    oh_t = (nt == lax.broadcasted_iota(jnp.int32, (BR, 8), 1).astype(_f32)
            ).astype(_f32)
    te = _mm(oh_t, ttab[...])
    oh_c = (ci == lax.broadcasted_iota(jnp.int32, (BR, 24), 1).astype(_f32)
            ).astype(_f32)
    ce = _mm(oh_c, ctab[...])
    oh_v = (vi == lax.broadcasted_iota(jnp.int32, (BR, 8), 1).astype(_f32)
            ).astype(_f32)
    vee = _mm(oh_v, vtab[...])
    ve = jnp.where(nt == 3.0, ce, jnp.where(nt == 4.0, vee, 0.0))
    hout[...] = _mm(jnp.concatenate([te, ve], axis=1), iw[...]) + ib[...]


def _tc_init(meta, ttab, ctab, vtab, iw, ib):
    specs = [pl.BlockSpec((BR, 8), lambda i: (i, 0))]
    specs += [_full_spec(x.shape) for x in (ttab, ctab, vtab, iw, ib)]
    return pl.pallas_call(
        _tc_init_body,
        grid=(N // BR,),
        in_specs=specs,
        out_specs=_node_spec(),
        out_shape=jax.ShapeDtypeStruct((N, H), _f32),
    )(meta, ttab, ctab, vtab, iw, ib)


def _tc_final_body(h, gw, gb, emb, gco, gc_v):
    i = pl.program_id(0)

    @pl.when(i == 0)
    def _():
        gc_v[...] = _mm(h[0:8, :], gw[...]) + gb[...]

    hv = h[...]
    emb[:, 0:H] = hv
    emb[:, H:2 * H] = jnp.broadcast_to(gc_v[0:1, :], (BR, H))
    gco[...] = gc_v[...]


def _tc_final(h, gw, gb):
    return pl.pallas_call(
        _tc_final_body,
        grid=(N // BR,),
        in_specs=[_node_spec(), _full_spec((H, H)), _full_spec((1, H))],
        out_specs=[pl.BlockSpec((BR, 2 * H), lambda i: (i, 0)),
                   _full_spec((8, H))],
        out_shape=[jax.ShapeDtypeStruct((N, 2 * H), _f32),
                   jax.ShapeDtypeStruct((8, H), _f32)],
        scratch_shapes=[pltpu.VMEM((8, H), _f32)],
    )(h, gw, gb)


def kernel(node_type, coeff_idx, var_idx, src, dst, type_table, coeff_table,
           var_table, init_w, init_b, agg_comm_w, agg_comm_b, agg_bin_w,
           agg_bin_b, wih_comm, whh_comm, bih_comm, bhh_comm, wih_bin,
           whh_bin, bih_bin, bhh_bin, wih_leaf, whh_leaf, bih_leaf, bhh_leaf,
           ln_g, ln_b, glob_w, glob_b):
    i32 = jnp.int32
    src = src.astype(i32)
    dst = dst.astype(i32)

    start = jnp.searchsorted(dst, jnp.arange(N + 1, dtype=i32), side='left')
    start = start.astype(i32)
    counts = (start[1:] - start[:N]).astype(_f32)
    c0 = src[jnp.clip(start[:N], 0, E - 1)]
    c1 = src[jnp.clip(start[:N] + 1, 0, E - 1)]
    bounds = jnp.minimum(jnp.arange(33, dtype=i32) * NPW, N)
    eoff = start[bounds]
    eoffp = jnp.concatenate([eoff, jnp.full((15,), E, i32)])
    dstl = dst % NPW
    srcp = jnp.concatenate([src, jnp.zeros((CH,), i32)])
    dstlp = jnp.concatenate([dstl, jnp.full((CH,), DUMP, i32)])
    c0p = jnp.concatenate([c0, jnp.zeros((NPAD - N,), i32)])
    c1p = jnp.concatenate([c1, jnp.zeros((NPAD - N,), i32)])

    is_leaf = (counts == 0.0)
    is_comm = (node_type <= 1) & (~is_leaf)
    is_pow2 = (node_type == 2) & (counts == 2.0)
    invd = 1.0 / jnp.maximum(counts, 1.0)
    meta = jnp.stack([counts, invd,
                      is_leaf.astype(_f32), is_comm.astype(_f32),
                      is_pow2.astype(_f32),
                      jnp.zeros((N,), _f32), jnp.zeros((N,), _f32),
                      jnp.zeros((N,), _f32)], axis=1)

    meta_init = jnp.stack([node_type.astype(_f32), coeff_idx.astype(_f32),
                           var_idx.astype(_f32)] + [jnp.zeros((N,), _f32)] * 5,
                          axis=1)
    ttab = jnp.zeros((8, H), _f32).at[:6].set(type_table)
    ctab = jnp.zeros((24, H), _f32).at[:19].set(coeff_table)
    vtab = jnp.zeros((8, H), _f32).at[:4].set(var_table)

    rw = [agg_comm_w, agg_comm_b.reshape(1, H), agg_bin_w,
          agg_bin_b.reshape(1, H),
          wih_comm, whh_comm, bih_comm.reshape(1, 3 * H),
          bhh_comm.reshape(1, 3 * H),
          wih_bin, whh_bin, bih_bin.reshape(1, 3 * H),
          bhh_bin.reshape(1, 3 * H),
          wih_leaf, whh_leaf, bih_leaf.reshape(1, 3 * H),
          bhh_leaf.reshape(1, 3 * H),
          ln_g.reshape(1, H), ln_b.reshape(1, H)]

    h = _tc_init(meta_init, ttab, ctab, vtab, init_w, init_b.reshape(1, H))
    for _r in range(NUM_ROUNDS):
        sm, mx, hc0, hc1 = _sc_agg(h, srcp, dstlp, eoffp, c0p, c1p)
        h = _tc_round(h, sm, mx, hc0, hc1, meta, *rw)

    emb, gc8 = _tc_final(h, glob_w, glob_b.reshape(1, H))
    return emb, gc8[0]
